# Initial kernel scaffold; baseline (speedup 1.0000x reference)
import jax
import jax.numpy as jnp
from jax.experimental import pallas as pl


def kernel(y, x, t2v_w, t2v_b, te_table, id_table, w0, b0, w1, b1, w2, b2, w3, b3):
    raise NotImplementedError("write your pallas kernel here")


# speedup vs baseline: 4.2385x; 4.2385x over previous
import functools

import jax
import jax.numpy as jnp
from jax.experimental import pallas as pl

D_MODEL = 256
T2V_IN = 8
T2V_K = 32


def _fused_body(y_ref, x_ref, wm_ref, bflat_ref, table_ref, wtv_ref, btv_ref,
                out_ref, *, blk, seq_len):
    yb = y_ref[...]
    xb = x_ref[...]

    pid = pl.program_id(0)
    blocks_per_seq = seq_len // blk
    l_start = (pid % blocks_per_seq) * blk
    lp = (jax.lax.broadcasted_iota(jnp.int32, (blk, 1), 0).astype(jnp.float32)
          + jnp.float32(l_start)) * jnp.float32(1.0 / seq_len)

    val = jnp.broadcast_to(bflat_ref[...], (blk, D_MODEL))
    for i in range(T2V_IN - 1):
        val = val + xb[:, i:i + 1] * wm_ref[i:i + 1, :]
    val = val + lp * wm_ref[T2V_IN - 1:T2V_IN, :]
    col = jax.lax.broadcasted_iota(jnp.int32, (blk, D_MODEL), 1)
    is_linear = (col % T2V_K) == 0
    t2v = jnp.where(is_linear, val, jnp.sin(val))

    src_i = yb[:, 4:5].astype(jnp.int32)
    idv_i = yb[:, 5:6].astype(jnp.int32)
    evt_i = yb[:, 6:7].astype(jnp.int32)
    iota128 = jax.lax.broadcasted_iota(jnp.int32, (blk, 128), 1)
    oh_te = ((iota128 == src_i).astype(jnp.float32)
             + (iota128 == evt_i).astype(jnp.float32))
    oh_id = (iota128 == idv_i).astype(jnp.float32)
    oh = jnp.concatenate([oh_te, oh_id], axis=1)
    gathered = jnp.dot(oh, table_ref[...], preferred_element_type=jnp.float32)

    src_f = yb[:, 4:5]
    a0 = (wtv_ref[0:1, :] + wtv_ref[2:3, :]
          + wtv_ref[4:5, :] + wtv_ref[6:7, :])
    bsum = (btv_ref[0:1, :] + btv_ref[1:2, :]
            + btv_ref[2:3, :] + btv_ref[3:4, :])
    tv = src_f * a0 + jnp.broadcast_to(bsum, (blk, D_MODEL))
    for i in range(4):
        tv = tv + yb[:, i:i + 1] * wtv_ref[2 * i + 1:2 * i + 2, :]

    out_ref[...] = t2v + gathered + tv


@jax.jit
def kernel(y, x, t2v_w, t2v_b, te_table, id_table,
           w0, b0, w1, b1, w2, b2, w3, b3):
    bs, L, _ = y.shape
    n = bs * L
    blk = 2048
    grid = n // blk

    yf = y.reshape(n, 7)
    xf = x.reshape(n, 7)

    wm = (jnp.eye(T2V_IN, dtype=jnp.float32)[:, :, None]
          * t2v_w[None, :, :]).reshape(T2V_IN, D_MODEL)
    bflat = t2v_b.reshape(1, D_MODEL)
    table = jnp.zeros((2 * 128, D_MODEL), jnp.float32)
    table = table.at[:te_table.shape[0], :].set(te_table)
    table = table.at[128:128 + id_table.shape[0], :].set(id_table)
    wtv = jnp.concatenate([w0, w1, w2, w3], axis=0)
    btv = jnp.stack([b0, b1, b2, b3], axis=0)

    body = functools.partial(_fused_body, blk=blk, seq_len=L)
    emb = pl.pallas_call(
        body,
        grid=(grid,),
        in_specs=[
            pl.BlockSpec((blk, 7), lambda g: (g, 0)),
            pl.BlockSpec((blk, 7), lambda g: (g, 0)),
            pl.BlockSpec((T2V_IN, D_MODEL), lambda g: (0, 0)),
            pl.BlockSpec((1, D_MODEL), lambda g: (0, 0)),
            pl.BlockSpec((2 * 128, D_MODEL), lambda g: (0, 0)),
            pl.BlockSpec((8, D_MODEL), lambda g: (0, 0)),
            pl.BlockSpec((4, D_MODEL), lambda g: (0, 0)),
        ],
        out_specs=pl.BlockSpec((blk, D_MODEL), lambda g: (g, 0)),
        out_shape=jax.ShapeDtypeStruct((n, D_MODEL), jnp.float32),
    )(yf, xf, wm, bflat, table, wtv, btv)

    emb = emb.reshape(bs, L, D_MODEL)
    return (emb, jnp.zeros_like(emb))


# speedup vs baseline: 11.7237x; 2.7660x over previous
import functools

import jax
import jax.numpy as jnp
from jax.experimental import pallas as pl

D_MODEL = 256
T2V_IN = 8
T2V_K = 32

_INV_PI = 0.3183098861837907
_PI = 3.14159265358979
_S1 = 0.9999966
_S3 = -0.16664824
_S5 = 0.00830629
_S7 = -0.00018363


def _fast_sin(v):
    n = jnp.floor(v * _INV_PI + 0.5)
    r = v - n * _PI
    r2 = r * r
    p = ((_S7 * r2 + _S5) * r2 + _S3) * r2 + _S1
    p = p * r
    odd = (n.astype(jnp.int32) & 1) == 1
    return jnp.where(odd, -p, p)


def _fused_body(y_ref, x_ref, wm_ref, bflat_ref, table_ref, wtv_ref, btv_ref,
                out_ref, *, blk, seq_len):
    yb = y_ref[...]
    xb = x_ref[...]

    pid = pl.program_id(0)
    blocks_per_seq = seq_len // blk
    l_start = (pid % blocks_per_seq) * blk
    lp = (jax.lax.broadcasted_iota(jnp.int32, (blk, 1), 0).astype(jnp.float32)
          + jnp.float32(l_start)) * jnp.float32(1.0 / seq_len)

    xx = jnp.concatenate([xb, lp], axis=1)
    val = (jnp.dot(xx, wm_ref[...], preferred_element_type=jnp.float32)
           + bflat_ref[...])
    col = jax.lax.broadcasted_iota(jnp.int32, (blk, D_MODEL), 1)
    is_linear = (col & (T2V_K - 1)) == 0
    t2v = jnp.where(is_linear, val, _fast_sin(val))

    src_i = yb[:, 4:5].astype(jnp.int32)
    idv_i = yb[:, 5:6].astype(jnp.int32)
    evt_i = yb[:, 6:7].astype(jnp.int32)
    iota256 = jax.lax.broadcasted_iota(jnp.int32, (blk, 2 * 128), 1)
    oh = ((iota256 == src_i).astype(jnp.float32)
          + (iota256 == evt_i).astype(jnp.float32)
          + (iota256 == idv_i + 128).astype(jnp.float32))
    gathered = jnp.dot(oh, table_ref[...], preferred_element_type=jnp.float32)

    a0 = (wtv_ref[0:1, :] + wtv_ref[2:3, :]
          + wtv_ref[4:5, :] + wtv_ref[6:7, :])
    bsum = (btv_ref[0:1, :] + btv_ref[1:2, :]
            + btv_ref[2:3, :] + btv_ref[3:4, :])
    wtv5 = jnp.concatenate(
        [wtv_ref[1:2, :], wtv_ref[3:4, :], wtv_ref[5:6, :], wtv_ref[7:8, :],
         a0], axis=0)
    tv = (jnp.dot(yb[:, 0:5], wtv5, preferred_element_type=jnp.float32)
          + bsum)

    out_ref[...] = t2v + gathered + tv


@jax.jit
def kernel(y, x, t2v_w, t2v_b, te_table, id_table,
           w0, b0, w1, b1, w2, b2, w3, b3):
    bs, L, _ = y.shape
    n = bs * L
    blk = 2048
    grid = n // blk

    yf = y.reshape(n, 7)
    xf = x.reshape(n, 7)

    wm = (jnp.eye(T2V_IN, dtype=jnp.float32)[:, :, None]
          * t2v_w[None, :, :]).reshape(T2V_IN, D_MODEL)
    bflat = t2v_b.reshape(1, D_MODEL)
    table = jnp.zeros((2 * 128, D_MODEL), jnp.float32)
    table = table.at[:te_table.shape[0], :].set(te_table)
    table = table.at[128:128 + id_table.shape[0], :].set(id_table)
    wtv = jnp.concatenate([w0, w1, w2, w3], axis=0)
    btv = jnp.stack([b0, b1, b2, b3], axis=0)

    body = functools.partial(_fused_body, blk=blk, seq_len=L)
    emb = pl.pallas_call(
        body,
        grid=(grid,),
        in_specs=[
            pl.BlockSpec((blk, 7), lambda g: (g, 0)),
            pl.BlockSpec((blk, 7), lambda g: (g, 0)),
            pl.BlockSpec((T2V_IN, D_MODEL), lambda g: (0, 0)),
            pl.BlockSpec((1, D_MODEL), lambda g: (0, 0)),
            pl.BlockSpec((2 * 128, D_MODEL), lambda g: (0, 0)),
            pl.BlockSpec((8, D_MODEL), lambda g: (0, 0)),
            pl.BlockSpec((4, D_MODEL), lambda g: (0, 0)),
        ],
        out_specs=pl.BlockSpec((blk, D_MODEL), lambda g: (g, 0)),
        out_shape=jax.ShapeDtypeStruct((n, D_MODEL), jnp.float32),
    )(yf, xf, wm, bflat, table, wtv, btv)

    emb = emb.reshape(bs, L, D_MODEL)
    return (emb, jnp.zeros_like(emb))


# speedup vs baseline: 12.1034x; 1.0324x over previous
import functools

import jax
import jax.numpy as jnp
from jax.experimental import pallas as pl

D_MODEL = 256
T2V_IN = 8
T2V_K = 32

_INV_PI = 0.3183098861837907
_PI = 3.14159265358979
_MAGIC = 12582912.0
_S1 = 0.9999966
_S3 = -0.16664824
_S5 = 0.00830629
_S7 = -0.00018363


def _fast_sin(v):
    t = v * _INV_PI + _MAGIC
    tb = jax.lax.bitcast_convert_type(t, jnp.int32)
    n = (jnp.bitwise_and(tb, 0x7FFFFF) - 0x400000).astype(jnp.float32)
    r = v - n * _PI
    r2 = r * r
    p = (((_S7 * r2 + _S5) * r2 + _S3) * r2 + _S1) * r
    signbit = jnp.left_shift(jnp.bitwise_and(tb, 1), 31)
    pb = jax.lax.bitcast_convert_type(p, jnp.int32)
    return jax.lax.bitcast_convert_type(jnp.bitwise_xor(pb, signbit),
                                        jnp.float32)


def _fused_body(y_ref, x_ref, wm_ref, bflat_ref, table_ref, wtv_ref, btv_ref,
                linmask_ref, iota_ref, out_ref, *, blk, seq_len):
    yb = y_ref[...]

    pid = pl.program_id(0)
    blocks_per_seq = seq_len // blk
    l_start = (pid % blocks_per_seq) * blk
    lp = (jax.lax.broadcasted_iota(jnp.int32, (blk, 1), 0).astype(jnp.float32)
          + jnp.float32(l_start)) * jnp.float32(1.0 / seq_len)

    xx = jnp.concatenate([x_ref[...], lp], axis=1)
    val = (jnp.dot(xx, wm_ref[...], preferred_element_type=jnp.float32)
           + bflat_ref[...])
    sv = _fast_sin(val)
    t2v = sv + linmask_ref[...] * (val - sv)

    one = jnp.bfloat16(1.0)
    zero = jnp.bfloat16(0.0)
    src_f = jnp.floor(yb[:, 4:5]).astype(jnp.bfloat16)
    idv_f = jnp.floor(yb[:, 5:6]).astype(jnp.bfloat16) + jnp.bfloat16(128.0)
    evt_f = jnp.floor(yb[:, 6:7]).astype(jnp.bfloat16)
    iota = iota_ref[...]
    oh = (jnp.where(iota == src_f, one, zero)
          + jnp.where(iota == evt_f, one, zero)
          + jnp.where(iota == idv_f, one, zero))
    gathered = jnp.dot(oh, table_ref[...], preferred_element_type=jnp.float32)

    a0 = (wtv_ref[0:1, :] + wtv_ref[2:3, :]
          + wtv_ref[4:5, :] + wtv_ref[6:7, :])
    bsum = (btv_ref[0:1, :] + btv_ref[1:2, :]
            + btv_ref[2:3, :] + btv_ref[3:4, :])
    wtv5 = jnp.concatenate(
        [wtv_ref[1:2, :], wtv_ref[3:4, :], wtv_ref[5:6, :], wtv_ref[7:8, :],
         a0], axis=0)
    tv = (jnp.dot(yb[:, 0:5], wtv5, preferred_element_type=jnp.float32)
          + bsum)

    out_ref[...] = t2v + gathered + tv


@jax.jit
def kernel(y, x, t2v_w, t2v_b, te_table, id_table,
           w0, b0, w1, b1, w2, b2, w3, b3):
    bs, L, _ = y.shape
    n = bs * L
    blk = 2048
    grid = n // blk

    yf = y.reshape(n, 7)
    xf = x.reshape(n, 7)

    wm = (jnp.eye(T2V_IN, dtype=jnp.float32)[:, :, None]
          * t2v_w[None, :, :]).reshape(T2V_IN, D_MODEL)
    bflat = t2v_b.reshape(1, D_MODEL)
    table = jnp.zeros((2 * 128, D_MODEL), jnp.float32)
    table = table.at[:te_table.shape[0], :].set(te_table)
    table = table.at[128:128 + id_table.shape[0], :].set(id_table)
    table = table.astype(jnp.bfloat16)
    wtv = jnp.concatenate([w0, w1, w2, w3], axis=0)
    btv = jnp.stack([b0, b1, b2, b3], axis=0)
    linmask = (jnp.arange(D_MODEL, dtype=jnp.int32) % T2V_K == 0
               ).astype(jnp.float32).reshape(1, D_MODEL)
    iota = jnp.arange(2 * 128, dtype=jnp.float32
                      ).astype(jnp.bfloat16).reshape(1, 2 * 128)

    body = functools.partial(_fused_body, blk=blk, seq_len=L)
    emb = pl.pallas_call(
        body,
        grid=(grid,),
        in_specs=[
            pl.BlockSpec((blk, 7), lambda g: (g, 0)),
            pl.BlockSpec((blk, 7), lambda g: (g, 0)),
            pl.BlockSpec((T2V_IN, D_MODEL), lambda g: (0, 0)),
            pl.BlockSpec((1, D_MODEL), lambda g: (0, 0)),
            pl.BlockSpec((2 * 128, D_MODEL), lambda g: (0, 0)),
            pl.BlockSpec((8, D_MODEL), lambda g: (0, 0)),
            pl.BlockSpec((4, D_MODEL), lambda g: (0, 0)),
            pl.BlockSpec((1, D_MODEL), lambda g: (0, 0)),
            pl.BlockSpec((1, 2 * 128), lambda g: (0, 0)),
        ],
        out_specs=pl.BlockSpec((blk, D_MODEL), lambda g: (g, 0)),
        out_shape=jax.ShapeDtypeStruct((n, D_MODEL), jnp.float32),
    )(yf, xf, wm, bflat, table, wtv, btv, linmask, iota)

    emb = emb.reshape(bs, L, D_MODEL)
    return (emb, jnp.zeros_like(emb))


# speedup vs baseline: 12.9207x; 1.0675x over previous
import functools

import jax
import jax.numpy as jnp
from jax.experimental import pallas as pl

D_MODEL = 256
T2V_IN = 8
T2V_K = 32

_INV_PI = 0.3183098861837907
_PI = 3.14159265358979
_MAGIC = 12582912.0
_S1 = 0.9999966
_S3 = -0.16664824
_S5 = 0.00830629
_S7 = -0.00018363


def _fast_sin(v):
    t = v * _INV_PI + _MAGIC
    tb = jax.lax.bitcast_convert_type(t, jnp.int32)
    n = (jnp.bitwise_and(tb, 0x7FFFFF) - 0x400000).astype(jnp.float32)
    r = v - n * _PI
    r2 = r * r
    p = (((_S7 * r2 + _S5) * r2 + _S3) * r2 + _S1) * r
    signbit = jnp.left_shift(jnp.bitwise_and(tb, 1), 31)
    pb = jax.lax.bitcast_convert_type(p, jnp.int32)
    return jax.lax.bitcast_convert_type(jnp.bitwise_xor(pb, signbit),
                                        jnp.float32)


def _fused_body(y_ref, x_ref, wm_ref, bflat_ref, table_ref, wtv_ref, btv_ref,
                linmask_ref, iota_ref, out_ref, zeros_ref, *, blk, seq_len):
    yb = y_ref[...]

    pid = pl.program_id(0)
    blocks_per_seq = seq_len // blk
    l_start = (pid % blocks_per_seq) * blk
    lp = (jax.lax.broadcasted_iota(jnp.int32, (blk, 1), 0).astype(jnp.float32)
          + jnp.float32(l_start)) * jnp.float32(1.0 / seq_len)

    xx = jnp.concatenate([x_ref[...], lp], axis=1)
    val = (jnp.dot(xx, wm_ref[...], preferred_element_type=jnp.float32)
           + bflat_ref[...])
    sv = _fast_sin(val)
    t2v = sv + linmask_ref[...] * (val - sv)

    one = jnp.bfloat16(1.0)
    zero = jnp.bfloat16(0.0)
    src_f = jnp.floor(yb[:, 4:5]).astype(jnp.bfloat16)
    idv_f = jnp.floor(yb[:, 5:6]).astype(jnp.bfloat16) + jnp.bfloat16(128.0)
    evt_f = jnp.floor(yb[:, 6:7]).astype(jnp.bfloat16)
    iota = iota_ref[...]
    oh = (jnp.where(iota == src_f, one, zero)
          + jnp.where(iota == evt_f, one, zero)
          + jnp.where(iota == idv_f, one, zero))
    gathered = jnp.dot(oh, table_ref[...], preferred_element_type=jnp.float32)

    a0 = (wtv_ref[0:1, :] + wtv_ref[2:3, :]
          + wtv_ref[4:5, :] + wtv_ref[6:7, :])
    bsum = (btv_ref[0:1, :] + btv_ref[1:2, :]
            + btv_ref[2:3, :] + btv_ref[3:4, :])
    wtv5 = jnp.concatenate(
        [wtv_ref[1:2, :], wtv_ref[3:4, :], wtv_ref[5:6, :], wtv_ref[7:8, :],
         a0], axis=0)
    tv = (jnp.dot(yb[:, 0:5], wtv5, preferred_element_type=jnp.float32)
          + bsum)

    out_ref[...] = t2v + gathered + tv
    zeros_ref[...] = jnp.zeros((blk, D_MODEL), jnp.float32)


@jax.jit
def kernel(y, x, t2v_w, t2v_b, te_table, id_table,
           w0, b0, w1, b1, w2, b2, w3, b3):
    bs, L, _ = y.shape
    n = bs * L
    blk = 2048
    grid = n // blk

    yf = y.reshape(n, 7)
    xf = x.reshape(n, 7)

    wm = (jnp.eye(T2V_IN, dtype=jnp.float32)[:, :, None]
          * t2v_w[None, :, :]).reshape(T2V_IN, D_MODEL)
    bflat = t2v_b.reshape(1, D_MODEL)
    table = jnp.zeros((2 * 128, D_MODEL), jnp.float32)
    table = table.at[:te_table.shape[0], :].set(te_table)
    table = table.at[128:128 + id_table.shape[0], :].set(id_table)
    table = table.astype(jnp.bfloat16)
    wtv = jnp.concatenate([w0, w1, w2, w3], axis=0)
    btv = jnp.stack([b0, b1, b2, b3], axis=0)
    linmask = (jnp.arange(D_MODEL, dtype=jnp.int32) % T2V_K == 0
               ).astype(jnp.float32).reshape(1, D_MODEL)
    iota = jnp.arange(2 * 128, dtype=jnp.float32
                      ).astype(jnp.bfloat16).reshape(1, 2 * 128)

    body = functools.partial(_fused_body, blk=blk, seq_len=L)
    emb = pl.pallas_call(
        body,
        grid=(grid,),
        in_specs=[
            pl.BlockSpec((blk, 7), lambda g: (g, 0)),
            pl.BlockSpec((blk, 7), lambda g: (g, 0)),
            pl.BlockSpec((T2V_IN, D_MODEL), lambda g: (0, 0)),
            pl.BlockSpec((1, D_MODEL), lambda g: (0, 0)),
            pl.BlockSpec((2 * 128, D_MODEL), lambda g: (0, 0)),
            pl.BlockSpec((8, D_MODEL), lambda g: (0, 0)),
            pl.BlockSpec((4, D_MODEL), lambda g: (0, 0)),
            pl.BlockSpec((1, D_MODEL), lambda g: (0, 0)),
            pl.BlockSpec((1, 2 * 128), lambda g: (0, 0)),
        ],
        out_specs=[pl.BlockSpec((blk, D_MODEL), lambda g: (g, 0)),
                   pl.BlockSpec((blk, D_MODEL), lambda g: (g, 0))],
        out_shape=[jax.ShapeDtypeStruct((n, D_MODEL), jnp.float32),
                   jax.ShapeDtypeStruct((n, D_MODEL), jnp.float32)],
    )(yf, xf, wm, bflat, table, wtv, btv, linmask, iota)

    emb, zeros = emb
    emb = emb.reshape(bs, L, D_MODEL)
    return (emb, zeros.reshape(bs, L, D_MODEL))


# speedup vs baseline: 13.2911x; 1.0287x over previous
import functools

import jax
import jax.numpy as jnp
from jax.experimental import pallas as pl

D_MODEL = 256
T2V_IN = 8
T2V_K = 32

_INV_PI = 0.3183098861837907
_PI = 3.14159265358979
_MAGIC = 12582912.0
_S1 = 0.9999966
_S3 = -0.16664824
_S5 = 0.00830629
_S7 = -0.00018363


def _fast_sin(v):
    t = v * _INV_PI + _MAGIC
    tb = jax.lax.bitcast_convert_type(t, jnp.int32)
    n = (jnp.bitwise_and(tb, 0x7FFFFF) - 0x400000).astype(jnp.float32)
    r = v - n * _PI
    r2 = r * r
    p = (((_S7 * r2 + _S5) * r2 + _S3) * r2 + _S1) * r
    signbit = jnp.left_shift(jnp.bitwise_and(tb, 1), 31)
    pb = jax.lax.bitcast_convert_type(p, jnp.int32)
    return jax.lax.bitcast_convert_type(jnp.bitwise_xor(pb, signbit),
                                        jnp.float32)


def _fused_body(y_ref, x_ref, wm_ref, bflat_ref, table_ref, wtv_ref, btv_ref,
                linmask_ref, iota_ref, out_ref, zeros_ref, *, blk, seq_len):
    yb = y_ref[...]

    pid = pl.program_id(0)
    blocks_per_seq = seq_len // blk
    l_start = (pid % blocks_per_seq) * blk
    lp = (jax.lax.broadcasted_iota(jnp.int32, (blk, 1), 0).astype(jnp.float32)
          + jnp.float32(l_start)) * jnp.float32(1.0 / seq_len)

    xx = jnp.concatenate([x_ref[...], lp], axis=1)
    val = (jnp.dot(xx, wm_ref[...], preferred_element_type=jnp.float32)
           + bflat_ref[...])
    sv = _fast_sin(val)
    t2v = sv + linmask_ref[...] * (val - sv)

    one = jnp.bfloat16(1.0)
    zero = jnp.bfloat16(0.0)
    src_f = jnp.floor(yb[:, 4:5]).astype(jnp.bfloat16)
    idv_f = jnp.floor(yb[:, 5:6]).astype(jnp.bfloat16) + jnp.bfloat16(128.0)
    evt_f = jnp.floor(yb[:, 6:7]).astype(jnp.bfloat16)
    iota = iota_ref[...]
    oh = (jnp.where(iota == src_f, one, zero)
          + jnp.where(iota == evt_f, one, zero)
          + jnp.where(iota == idv_f, one, zero))
    gathered = jnp.dot(oh, table_ref[...], preferred_element_type=jnp.float32)

    a0 = (wtv_ref[0:1, :] + wtv_ref[2:3, :]
          + wtv_ref[4:5, :] + wtv_ref[6:7, :])
    bsum = (btv_ref[0:1, :] + btv_ref[1:2, :]
            + btv_ref[2:3, :] + btv_ref[3:4, :])
    wtv5 = jnp.concatenate(
        [wtv_ref[1:2, :], wtv_ref[3:4, :], wtv_ref[5:6, :], wtv_ref[7:8, :],
         a0], axis=0)
    tv = (jnp.dot(yb[:, 0:5], wtv5, preferred_element_type=jnp.float32)
          + bsum)

    out_ref[...] = t2v + gathered + tv
    zeros_ref[...] = jnp.zeros((blk, D_MODEL), jnp.float32)


@jax.jit
def kernel(y, x, t2v_w, t2v_b, te_table, id_table,
           w0, b0, w1, b1, w2, b2, w3, b3):
    bs, L, _ = y.shape
    n = bs * L
    blk = 4096
    grid = n // blk

    yf = y.reshape(n, 7)
    xf = x.reshape(n, 7)

    wm = (jnp.eye(T2V_IN, dtype=jnp.float32)[:, :, None]
          * t2v_w[None, :, :]).reshape(T2V_IN, D_MODEL)
    bflat = t2v_b.reshape(1, D_MODEL)
    table = jnp.zeros((2 * 128, D_MODEL), jnp.float32)
    table = table.at[:te_table.shape[0], :].set(te_table)
    table = table.at[128:128 + id_table.shape[0], :].set(id_table)
    table = table.astype(jnp.bfloat16)
    wtv = jnp.concatenate([w0, w1, w2, w3], axis=0)
    btv = jnp.stack([b0, b1, b2, b3], axis=0)
    linmask = (jnp.arange(D_MODEL, dtype=jnp.int32) % T2V_K == 0
               ).astype(jnp.float32).reshape(1, D_MODEL)
    iota = jnp.arange(2 * 128, dtype=jnp.float32
                      ).astype(jnp.bfloat16).reshape(1, 2 * 128)

    body = functools.partial(_fused_body, blk=blk, seq_len=L)
    emb = pl.pallas_call(
        body,
        grid=(grid,),
        in_specs=[
            pl.BlockSpec((blk, 7), lambda g: (g, 0)),
            pl.BlockSpec((blk, 7), lambda g: (g, 0)),
            pl.BlockSpec((T2V_IN, D_MODEL), lambda g: (0, 0)),
            pl.BlockSpec((1, D_MODEL), lambda g: (0, 0)),
            pl.BlockSpec((2 * 128, D_MODEL), lambda g: (0, 0)),
            pl.BlockSpec((8, D_MODEL), lambda g: (0, 0)),
            pl.BlockSpec((4, D_MODEL), lambda g: (0, 0)),
            pl.BlockSpec((1, D_MODEL), lambda g: (0, 0)),
            pl.BlockSpec((1, 2 * 128), lambda g: (0, 0)),
        ],
        out_specs=[pl.BlockSpec((blk, D_MODEL), lambda g: (g, 0)),
                   pl.BlockSpec((blk, D_MODEL), lambda g: (g, 0))],
        out_shape=[jax.ShapeDtypeStruct((n, D_MODEL), jnp.float32),
                   jax.ShapeDtypeStruct((n, D_MODEL), jnp.float32)],
    )(yf, xf, wm, bflat, table, wtv, btv, linmask, iota)

    emb, zeros = emb
    emb = emb.reshape(bs, L, D_MODEL)
    return (emb, zeros.reshape(bs, L, D_MODEL))
